# baseline (speedup 1.0000x reference)
import functools

import jax
import jax.numpy as jnp
from jax import lax
from jax.experimental import pallas as pl
from jax.experimental.pallas import tpu as pltpu
from jax.experimental.pallas import tpu_sc as plsc

D_MODEL_K = 2048
N_EXP = 64
N_TOK = 16384
BLOCK_T = 2048
N_WORKERS = 32
C_PER_W = N_TOK // N_WORKERS
L = 16


def _softmax_body(x_ref, w_ref, prob_ref, probt_ref):
    st = jax.lax.dot_general(
        w_ref[...], x_ref[...], (((1,), (1,)), ((), ())),
        preferred_element_type=jnp.float32)
    m = jnp.max(st, axis=0, keepdims=True)
    e = jnp.exp(st - m)
    denom = jnp.sum(e, axis=0, keepdims=True)
    pt = e * (1.0 / denom)
    probt_ref[...] = pt
    prob_ref[...] = pt.T


def _tc_prob(x, W):
    g = N_TOK // BLOCK_T
    return pl.pallas_call(
        _softmax_body,
        grid=(g,),
        in_specs=[
            pl.BlockSpec((BLOCK_T, D_MODEL_K), lambda i: (i, 0)),
            pl.BlockSpec((N_EXP, D_MODEL_K), lambda i: (0, 0)),
        ],
        out_specs=(
            pl.BlockSpec((BLOCK_T, N_EXP), lambda i: (i, 0)),
            pl.BlockSpec((N_EXP, BLOCK_T), lambda i: (0, i)),
        ),
        out_shape=(
            jax.ShapeDtypeStruct((N_TOK, N_EXP), jnp.float32),
            jax.ShapeDtypeStruct((N_EXP, N_TOK), jnp.float32),
        ),
    )(x, W)


@functools.partial(
    pl.kernel,
    mesh=plsc.VectorSubcoreMesh(core_axis_name="c", subcore_axis_name="s"),
    out_type=(
        jax.ShapeDtypeStruct((N_TOK,), jnp.int32),
        jax.ShapeDtypeStruct((N_TOK,), jnp.float32),
    ),
    scratch_types=[
        pltpu.VMEM((N_EXP, C_PER_W), jnp.float32),
        pltpu.VMEM((C_PER_W,), jnp.int32),
        pltpu.VMEM((C_PER_W,), jnp.float32),
    ],
)
def _sc_gate(probt_hbm, idx_hbm, p1_hbm, slab, idxv, p1v):
    wid = lax.axis_index("s") * 2 + lax.axis_index("c")
    base = wid * C_PER_W
    pltpu.sync_copy(probt_hbm.at[:, pl.ds(base, C_PER_W)], slab)

    def group(g, carry):
        off = g * L
        m = slab[0, pl.ds(off, L)]
        ei = jnp.zeros((L,), jnp.int32)
        for e in range(1, N_EXP):
            v = slab[e, pl.ds(off, L)]
            sel = v > m
            ei = jnp.where(sel, e, ei)
            m = jnp.maximum(m, v)
        idxv[pl.ds(off, L)] = ei
        p1v[pl.ds(off, L)] = m
        return carry

    lax.fori_loop(0, C_PER_W // L, group, 0)
    pltpu.sync_copy(idxv, idx_hbm.at[pl.ds(base, C_PER_W)])
    pltpu.sync_copy(p1v, p1_hbm.at[pl.ds(base, C_PER_W)])


def kernel(x, W):
    prob, probt = _tc_prob(x, W)
    idx, p1 = _sc_gate(probt)
    return (idx, p1, prob)


# speedup vs baseline: 1.3061x; 1.3061x over previous
import jax
import jax.numpy as jnp
from jax.experimental import pallas as pl

D_MODEL_K = 2048
N_EXP = 64
BLOCK_T = 2048


def _router_body(x_ref, w_ref, idx_ref, p1_ref, prob_ref):
    st = jax.lax.dot_general(
        w_ref[...], x_ref[...], (((1,), (1,)), ((), ())),
        preferred_element_type=jnp.float32)
    m = jnp.max(st, axis=0, keepdims=True)
    e = jnp.exp(st - m)
    denom = jnp.sum(e, axis=0, keepdims=True)
    r = 1.0 / denom
    prob_ref[...] = (e * r).T
    ii = jax.lax.broadcasted_iota(jnp.int32, st.shape, 0)
    idx_ref[0] = jnp.min(jnp.where(st == m, ii, N_EXP), axis=0, keepdims=True)
    p1_ref[0] = r


def kernel(x, W):
    n_tok = x.shape[0]
    g = n_tok // BLOCK_T
    out_shapes = (
        jax.ShapeDtypeStruct((g, 1, BLOCK_T), jnp.int32),
        jax.ShapeDtypeStruct((g, 1, BLOCK_T), jnp.float32),
        jax.ShapeDtypeStruct((n_tok, N_EXP), jnp.float32),
    )
    idx, p1, prob = pl.pallas_call(
        _router_body,
        grid=(g,),
        in_specs=[
            pl.BlockSpec((BLOCK_T, D_MODEL_K), lambda i: (i, 0)),
            pl.BlockSpec((N_EXP, D_MODEL_K), lambda i: (0, 0)),
        ],
        out_specs=(
            pl.BlockSpec((1, 1, BLOCK_T), lambda i: (i, 0, 0)),
            pl.BlockSpec((1, 1, BLOCK_T), lambda i: (i, 0, 0)),
            pl.BlockSpec((BLOCK_T, N_EXP), lambda i: (i, 0)),
        ),
        out_shape=out_shapes,
    )(x, W)
    return (idx.reshape(n_tok), p1.reshape(n_tok), prob)
